# baseline (speedup 1.0000x reference)
import functools

import jax
import jax.numpy as jnp
from jax import lax
from jax.experimental import pallas as pl
from jax.experimental.pallas import tpu as pltpu
from jax.experimental.pallas import tpu_sc as plsc

_NC = 2
_NS = 16
_NW = _NC * _NS
_CHUNK = 128


@functools.lru_cache(maxsize=None)
def _make_gather(n, d, nchunk):
    per_w = n // _NW
    mesh = plsc.VectorSubcoreMesh(core_axis_name="c", subcore_axis_name="s")

    def body(idx_hbm, table_hbm, out_hbm, idx_v, rows_v, g_sem):
        wid = lax.axis_index("s") * _NC + lax.axis_index("c")
        pltpu.sync_copy(idx_hbm.at[wid], idx_v)
        base = wid * per_w

        def step(j, carry):
            pltpu.async_copy(table_hbm.at[idx_v.at[j]], rows_v, g_sem).wait()
            pltpu.sync_copy(rows_v, out_hbm.at[pl.ds(base + j * _CHUNK, _CHUNK)])
            return carry

        lax.fori_loop(0, nchunk, step, 0)

    return pl.kernel(
        body,
        out_type=jax.ShapeDtypeStruct((n, d), jnp.float32),
        mesh=mesh,
        scratch_types=[
            pltpu.VMEM((nchunk, _CHUNK), jnp.int32),
            pltpu.VMEM((_CHUNK, d), jnp.float32),
            pltpu.SemaphoreType.DMA,
        ],
        compiler_params=pltpu.CompilerParams(use_tc_tiling_on_sc=False),
    )


def kernel(input_ids, W_emb):
    b, l = input_ids.shape
    v, d = W_emb.shape
    n = b * l
    nchunk = n // (_NW * _CHUNK)
    idx = input_ids.reshape(_NW, nchunk, _CHUNK).astype(jnp.int32)
    out = _make_gather(n, d, nchunk)(idx, W_emb)
    return out.reshape(b, l, d)


# speedup vs baseline: 1.1111x; 1.1111x over previous
import functools

import jax
import jax.numpy as jnp
from jax import lax
from jax.experimental import pallas as pl
from jax.experimental.pallas import tpu as pltpu
from jax.experimental.pallas import tpu_sc as plsc

_NC = 2
_NS = 16
_NW = _NC * _NS
_CHUNK = 128
_K = 2
_SLOTS = 4


@functools.lru_cache(maxsize=None)
def _make_gather(n, d, nchunk):
    per_w = n // _NW
    ngroups = nchunk // _K
    assert nchunk % _K == 0 and (ngroups - 4) % 4 == 0 and ngroups >= 8
    mesh = plsc.VectorSubcoreMesh(core_axis_name="c", subcore_axis_name="s")

    def body(idx_hbm, table_hbm, out_hbm, idx_v, rows_v,
             g0, g1, g2, g3, o0, o1, o2, o3):
        gsems = (g0, g1, g2, g3)
        osems = (o0, o1, o2, o3)
        wid = lax.axis_index("s") * _NC + lax.axis_index("c")
        pltpu.sync_copy(idx_hbm.at[wid], idx_v)
        base = wid * per_w

        def g_desc(g, s, b):
            c = g * _K + b
            return pltpu.make_async_copy(
                table_hbm.at[idx_v.at[c]], rows_v.at[s * _K + b], gsems[s])

        def o_desc(g, s, b):
            c = g * _K + b
            return pltpu.make_async_copy(
                rows_v.at[s * _K + b],
                out_hbm.at[pl.ds(base + c * _CHUNK, _CHUNK)], osems[s])

        def gfire(g, s):
            for b in range(_K):
                g_desc(g, s, b).start()

        def gdrain(g, s):
            for b in range(_K):
                g_desc(g, s, b).wait()

        def ofire(g, s):
            for b in range(_K):
                o_desc(g, s, b).start()

        def odrain(g, s):
            for b in range(_K):
                o_desc(g, s, b).wait()

        def part(g, s):
            odrain(g - 2, (s + 2) % _SLOTS)
            gfire(g + 2, (s + 2) % _SLOTS)
            gdrain(g, s)
            ofire(g, s)

        gfire(0, 0)
        gfire(1, 1)
        gfire(2, 2)
        gdrain(0, 0)
        ofire(0, 0)
        gfire(3, 3)
        gdrain(1, 1)
        ofire(1, 1)

        def step(i, carry):
            gbase = 4 * i + 2
            for q in range(4):
                part(gbase + q, (2 + q) % _SLOTS)
            return carry

        lax.fori_loop(0, (ngroups - 4) // 4, step, 0)

        ge = ngroups - 2
        odrain(ge - 2, 0)
        gdrain(ge, 2)
        ofire(ge, 2)
        odrain(ge - 1, 1)
        gdrain(ge + 1, 3)
        ofire(ge + 1, 3)
        odrain(ge, 2)
        odrain(ge + 1, 3)

    return pl.kernel(
        body,
        out_type=jax.ShapeDtypeStruct((n, d), jnp.float32),
        mesh=mesh,
        scratch_types=(
            [pltpu.VMEM((nchunk, _CHUNK), jnp.int32),
             pltpu.VMEM((_SLOTS * _K, _CHUNK, d), jnp.float32)]
            + [pltpu.SemaphoreType.DMA] * 8
        ),
        compiler_params=pltpu.CompilerParams(use_tc_tiling_on_sc=False),
    )


def kernel(input_ids, W_emb):
    b, l = input_ids.shape
    v, d = W_emb.shape
    n = b * l
    nchunk = n // (_NW * _CHUNK)
    idx = input_ids.reshape(_NW, nchunk, _CHUNK).astype(jnp.int32)
    out = _make_gather(n, d, nchunk)(idx, W_emb)
    return out.reshape(b, l, d)


# speedup vs baseline: 1.4873x; 1.3386x over previous
import functools

import jax
import jax.numpy as jnp
from jax import lax
from jax.experimental import pallas as pl
from jax.experimental.pallas import tpu as pltpu
from jax.experimental.pallas import tpu_sc as plsc

_NC = 2
_NS = 16
_NW = _NC * _NS
_CHUNK = 128
_K = 2
_SLOTS = 4


@functools.lru_cache(maxsize=None)
def _make_gather(n, d, nchunk):
    per_w = n // _NW
    ngroups = nchunk // _K
    assert nchunk % _K == 0 and (ngroups - 4) % 4 == 0 and ngroups >= 8
    mesh = plsc.VectorSubcoreMesh(core_axis_name="c", subcore_axis_name="s")

    def body(idx_hbm, table_hbm, out_hbm, idx_v, rows_v,
             g0, g1, g2, g3, o0, o1, o2, o3):
        gsems = (g0, g1, g2, g3)
        osems = (o0, o1, o2, o3)
        wid = lax.axis_index("s") * _NC + lax.axis_index("c")
        pltpu.sync_copy(idx_hbm.at[wid], idx_v)
        base = wid * per_w

        def g_desc(g, s, b):
            c = g * _K + b
            return pltpu.make_async_copy(
                table_hbm.at[idx_v.at[c]], rows_v.at[s * _K + b], gsems[s])

        def o_desc(g, s, b):
            c = g * _K + b
            return pltpu.make_async_copy(
                rows_v.at[s * _K + b],
                out_hbm.at[pl.ds(base + c * _CHUNK, _CHUNK), pl.ds(0, d)],
                osems[s])

        def gfire(g, s):
            for b in range(_K):
                g_desc(g, s, b).start()

        def gdrain(g, s):
            for b in range(_K):
                g_desc(g, s, b).wait()

        def ofire(g, s):
            for b in range(_K):
                o_desc(g, s, b).start()

        def odrain(g, s):
            for b in range(_K):
                o_desc(g, s, b).wait()

        def part(g, s):
            odrain(g - 2, (s + 2) % _SLOTS)
            gfire(g + 2, (s + 2) % _SLOTS)
            gdrain(g, s)
            ofire(g, s)

        gfire(0, 0)
        gfire(1, 1)
        gfire(2, 2)
        gdrain(0, 0)
        ofire(0, 0)
        gfire(3, 3)
        gdrain(1, 1)
        ofire(1, 1)

        def step(i, carry):
            gbase = 4 * i + 2
            for q in range(4):
                part(gbase + q, (2 + q) % _SLOTS)
            return carry

        lax.fori_loop(0, (ngroups - 4) // 4, step, 0)

        ge = ngroups - 2
        odrain(ge - 2, 0)
        gdrain(ge, 2)
        ofire(ge, 2)
        odrain(ge - 1, 1)
        gdrain(ge + 1, 3)
        ofire(ge + 1, 3)
        odrain(ge, 2)
        odrain(ge + 1, 3)

    return pl.kernel(
        body,
        out_type=jax.ShapeDtypeStruct((n, 2 * d), jnp.float32),
        mesh=mesh,
        scratch_types=(
            [pltpu.VMEM((nchunk, _CHUNK), jnp.int32),
             pltpu.VMEM((_SLOTS * _K, _CHUNK, d), jnp.float32)]
            + [pltpu.SemaphoreType.DMA] * 8
        ),
        compiler_params=pltpu.CompilerParams(use_tc_tiling_on_sc=False),
    )


def kernel(input_ids, W_emb):
    b, l = input_ids.shape
    v, d = W_emb.shape
    n = b * l
    nchunk = n // (_NW * _CHUNK)
    idx = input_ids.reshape(_NW, nchunk, _CHUNK).astype(jnp.int32)
    out = _make_gather(n, d, nchunk)(idx, W_emb)
    return out[:, :d].reshape(b, l, d)
